# baseline (speedup 1.0000x reference)
import jax
import jax.numpy as jnp
from jax import lax
from jax.experimental import pallas as pl

_MAX_PER_CLASS = 8
_MAX_TOTAL = 8
_SCORE_TH = 0.5


def _nms_body(boxes_ref, scores_ref, ob_ref, os_ref, oc_ref, ov_ref):
    bt = boxes_ref[0]
    y1 = jnp.minimum(bt[0:1], bt[2:3])
    x1 = jnp.minimum(bt[1:2], bt[3:4])
    y2 = jnp.maximum(bt[0:1], bt[2:3])
    x2 = jnp.maximum(bt[1:2], bt[3:4])
    area = (y2 - y1) * (x2 - x1)

    s = scores_ref[0]
    c, n = s.shape
    s = jnp.where(s > _SCORE_TH, s, -1.0)
    col = lax.broadcasted_iota(jnp.int32, (c, n), 1)

    sel = [[], [], [], [], []]
    for _ in range(_MAX_PER_CLASS):
        best = jnp.max(s, axis=1, keepdims=True)
        idx = jnp.min(jnp.where(s == best, col, n), axis=1,
                      keepdims=True)
        onehot = col == idx
        by1 = jnp.sum(jnp.where(onehot, y1, 0.0), axis=1, keepdims=True)
        bx1 = jnp.sum(jnp.where(onehot, x1, 0.0), axis=1, keepdims=True)
        by2 = jnp.sum(jnp.where(onehot, y2, 0.0), axis=1, keepdims=True)
        bx2 = jnp.sum(jnp.where(onehot, x2, 0.0), axis=1, keepdims=True)
        barea = (by2 - by1) * (bx2 - bx1)
        valid = best > 0.0
        for acc, v in zip(sel, (jnp.where(valid, best, 0.0), by1, bx1,
                                by2, bx2)):
            acc.append(v)
        inter = (jnp.maximum(jnp.minimum(y2, by2) - jnp.maximum(y1, by1), 0.0)
                 * (jnp.minimum(x2, bx2) - jnp.maximum(x1, bx1)))
        s = jnp.where((inter * 3.0 > area + barea) & valid, -1.0, s)

    tabs = [jnp.concatenate(acc, axis=1) for acc in sel]
    ms = tabs[0]
    kmax = c * _MAX_PER_CLASS
    c8 = lax.broadcasted_iota(jnp.int32, (c, _MAX_PER_CLASS), 0)
    r8 = lax.broadcasted_iota(jnp.int32, (c, _MAX_PER_CLASS), 1)
    key = c8 * _MAX_PER_CLASS + r8

    o_s, o_cls, o_vd, o_rows = [], [], [], []
    for _ in range(_MAX_TOTAL):
        m = jnp.max(ms, keepdims=True)
        km = jnp.min(jnp.where(ms == m, key, kmax), keepdims=True)
        oh = key == km
        vld = m > 0.0
        o_s.append(jnp.where(vld, m, 0.0))
        o_cls.append(jnp.where(vld, km // _MAX_PER_CLASS, 0).astype(jnp.int32))
        o_vd.append(vld.astype(jnp.int32))
        row = [jnp.clip(jnp.where(vld, jnp.sum(jnp.where(oh, cm, 0.0),
                                               keepdims=True), 0.0), 0.0, 1.0)
               for cm in tabs[1:]]
        o_rows.append(jnp.concatenate(row, axis=1))
        ms = jnp.where(oh, -1.0, ms)

    ob_ref[...] = jnp.concatenate(o_rows, axis=0)[None]
    os_ref[...] = jnp.concatenate(o_s, axis=1)[None]
    oc_ref[...] = jnp.concatenate(o_cls, axis=1)[None]
    ov_ref[...] = jnp.sum(jnp.concatenate(o_vd, axis=1), axis=1,
                          keepdims=True)[None]


def kernel(boxes, scores):
    bsz, n, _, _ = boxes.shape
    c = scores.shape[-1]
    bxt = jnp.transpose(boxes.reshape(bsz, n, 4), (0, 2, 1))
    st = jnp.transpose(scores, (0, 2, 1))

    ob, os_, oc, ov = pl.pallas_call(
        _nms_body,
        grid=(bsz,),
        in_specs=[
            pl.BlockSpec((1, 4, n), lambda i: (i, 0, 0)),
            pl.BlockSpec((1, c, n), lambda i: (i, 0, 0)),
        ],
        out_specs=[
            pl.BlockSpec((1, _MAX_TOTAL, 4), lambda i: (i, 0, 0)),
            pl.BlockSpec((1, 1, _MAX_TOTAL), lambda i: (i, 0, 0)),
            pl.BlockSpec((1, 1, _MAX_TOTAL), lambda i: (i, 0, 0)),
            pl.BlockSpec((1, 1, 1), lambda i: (i, 0, 0)),
        ],
        out_shape=[
            jax.ShapeDtypeStruct((bsz, _MAX_TOTAL, 4), jnp.float32),
            jax.ShapeDtypeStruct((bsz, 1, _MAX_TOTAL), jnp.float32),
            jax.ShapeDtypeStruct((bsz, 1, _MAX_TOTAL), jnp.int32),
            jax.ShapeDtypeStruct((bsz, 1, 1), jnp.int32),
        ],
    )(bxt, st)
    return (ob, os_.reshape(bsz, _MAX_TOTAL), oc.reshape(bsz, _MAX_TOTAL),
            ov.reshape(bsz))


# speedup vs baseline: 1.1547x; 1.1547x over previous
import jax
import jax.numpy as jnp
from jax import lax
from jax.experimental import pallas as pl

_MAX_PER_CLASS = 8
_MAX_TOTAL = 8
_SCORE_TH = 0.5
_CBLK = 40


def _nms_body(boxes_ref, scores_ref, ts_ref, ty1_ref, tx1_ref, ty2_ref,
              tx2_ref):
    bt = boxes_ref[0]
    y1 = jnp.minimum(bt[0:1], bt[2:3])
    x1 = jnp.minimum(bt[1:2], bt[3:4])
    y2 = jnp.maximum(bt[0:1], bt[2:3])
    x2 = jnp.maximum(bt[1:2], bt[3:4])
    area = (y2 - y1) * (x2 - x1)

    s = scores_ref[0]
    cb, n = s.shape
    s = jnp.where(s > _SCORE_TH, s, -1.0)
    col = lax.broadcasted_iota(jnp.int32, (cb, n), 1)

    sel = [[], [], [], [], []]
    for _ in range(_MAX_PER_CLASS):
        best = jnp.max(s, axis=1, keepdims=True)
        idx = jnp.min(jnp.where(s == best, col, n), axis=1,
                      keepdims=True)
        onehot = col == idx
        by1 = jnp.sum(jnp.where(onehot, y1, 0.0), axis=1, keepdims=True)
        bx1 = jnp.sum(jnp.where(onehot, x1, 0.0), axis=1, keepdims=True)
        by2 = jnp.sum(jnp.where(onehot, y2, 0.0), axis=1, keepdims=True)
        bx2 = jnp.sum(jnp.where(onehot, x2, 0.0), axis=1, keepdims=True)
        barea = (by2 - by1) * (bx2 - bx1)
        valid = best > 0.0
        for acc, v in zip(sel, (jnp.where(valid, best, 0.0), by1, bx1,
                                by2, bx2)):
            acc.append(v)
        inter = (jnp.maximum(jnp.minimum(y2, by2) - jnp.maximum(y1, by1), 0.0)
                 * (jnp.minimum(x2, bx2) - jnp.maximum(x1, bx1)))
        s = jnp.where((inter * 3.0 > area + barea) & valid, -1.0, s)

    for ref, acc in zip((ts_ref, ty1_ref, tx1_ref, ty2_ref, tx2_ref), sel):
        ref[...] = jnp.concatenate(acc, axis=1)[None]


def _merge_body(ts_ref, ty1_ref, tx1_ref, ty2_ref, tx2_ref,
                ob_ref, os_ref, oc_ref, ov_ref):
    ms = ts_ref[0]
    c = ms.shape[0]
    kmax = c * _MAX_PER_CLASS
    c8 = lax.broadcasted_iota(jnp.int32, (c, _MAX_PER_CLASS), 0)
    r8 = lax.broadcasted_iota(jnp.int32, (c, _MAX_PER_CLASS), 1)
    key = c8 * _MAX_PER_CLASS + r8

    coords_t = (ty1_ref[0], tx1_ref[0], ty2_ref[0], tx2_ref[0])
    o_s, o_cls, o_vd, o_rows = [], [], [], []
    for _ in range(_MAX_TOTAL):
        m = jnp.max(ms, keepdims=True)
        km = jnp.min(jnp.where(ms == m, key, kmax), keepdims=True)
        oh = key == km
        vld = m > 0.0
        o_s.append(jnp.where(vld, m, 0.0))
        o_cls.append(jnp.where(vld, km // _MAX_PER_CLASS, 0).astype(jnp.int32))
        o_vd.append(vld.astype(jnp.int32))
        row = [jnp.clip(jnp.where(vld, jnp.sum(jnp.where(oh, cm, 0.0),
                                               keepdims=True), 0.0), 0.0, 1.0)
               for cm in coords_t]
        o_rows.append(jnp.concatenate(row, axis=1))
        ms = jnp.where(oh, -1.0, ms)

    ob_ref[...] = jnp.concatenate(o_rows, axis=0)[None]
    os_ref[...] = jnp.concatenate(o_s, axis=1)[None]
    oc_ref[...] = jnp.concatenate(o_cls, axis=1)[None]
    ov_ref[...] = jnp.sum(jnp.concatenate(o_vd, axis=1), axis=1,
                          keepdims=True)[None]


def kernel(boxes, scores):
    bsz, n, _, _ = boxes.shape
    c = scores.shape[-1]
    bxt = jnp.transpose(boxes.reshape(bsz, n, 4), (0, 2, 1))
    st = jnp.transpose(scores, (0, 2, 1))

    tab_shape = jax.ShapeDtypeStruct((bsz, c, _MAX_PER_CLASS), jnp.float32)
    tab_spec = pl.BlockSpec((1, _CBLK, _MAX_PER_CLASS),
                            lambda i, j: (i, j, 0))
    tabs = pl.pallas_call(
        _nms_body,
        grid=(bsz, c // _CBLK),
        in_specs=[
            pl.BlockSpec((1, 4, n), lambda i, j: (i, 0, 0)),
            pl.BlockSpec((1, _CBLK, n), lambda i, j: (i, j, 0)),
        ],
        out_specs=[tab_spec] * 5,
        out_shape=[tab_shape] * 5,
    )(bxt, st)

    full_tab_spec = pl.BlockSpec((1, c, _MAX_PER_CLASS), lambda i: (i, 0, 0))
    ob, os_, oc, ov = pl.pallas_call(
        _merge_body,
        grid=(bsz,),
        in_specs=[full_tab_spec] * 5,
        out_specs=[
            pl.BlockSpec((1, _MAX_TOTAL, 4), lambda i: (i, 0, 0)),
            pl.BlockSpec((1, 1, _MAX_TOTAL), lambda i: (i, 0, 0)),
            pl.BlockSpec((1, 1, _MAX_TOTAL), lambda i: (i, 0, 0)),
            pl.BlockSpec((1, 1, 1), lambda i: (i, 0, 0)),
        ],
        out_shape=[
            jax.ShapeDtypeStruct((bsz, _MAX_TOTAL, 4), jnp.float32),
            jax.ShapeDtypeStruct((bsz, 1, _MAX_TOTAL), jnp.float32),
            jax.ShapeDtypeStruct((bsz, 1, _MAX_TOTAL), jnp.int32),
            jax.ShapeDtypeStruct((bsz, 1, 1), jnp.int32),
        ],
    )(*tabs)
    return (ob, os_.reshape(bsz, _MAX_TOTAL), oc.reshape(bsz, _MAX_TOTAL),
            ov.reshape(bsz))
